# baseline (speedup 1.0000x reference)
import functools

import jax
import jax.numpy as jnp
from jax import lax
from jax.experimental import pallas as pl
from jax.experimental.pallas import tpu as pltpu
from jax.experimental.pallas import tpu_sc as plsc

_B = 16384
_F = 26
_VOCAB = 100000
_DIM = 32
_LANE = 128
_RPL = _LANE // _DIM

_NC = 2
_NS = 16
_NW = _NC * _NS
_ROWS_PER_W = _B // _NW
_CB = 4
_CHUNK_IDX = _CB * _F
_NCHUNKS = _ROWS_PER_W // _CB
_NBUF = 4


def _sc_body(line_hbm, colb_hbm, table_hbm, out_hbm,
             line_v, colb_bufs, bufs, out_v, sems):
    wid = lax.axis_index("s") * _NC + lax.axis_index("c")

    pltpu.sync_copy(line_hbm.at[wid], line_v)

    def start(chunk, k):
        pltpu.async_copy(table_hbm.at[line_v.at[chunk]], bufs[k], sems[k])
        pltpu.async_copy(colb_hbm.at[wid, chunk], colb_bufs[k], sems[k])

    def wait(chunk, k):
        pltpu.make_async_copy(
            table_hbm.at[line_v.at[chunk]], bufs[k], sems[k]).wait()
        pltpu.make_async_copy(
            colb_hbm.at[wid, chunk], colb_bufs[k], sems[k]).wait()

    def accum(chunk, k):
        buf = bufs[k]
        colb = colb_bufs[k]
        for lb in range(_CB):
            base = lb * _F
            acc0 = None
            acc1 = None
            for f in range(_F):
                j = base + f
                rowv = jnp.full((16,), j, jnp.int32)
                colv = colb[j, :]
                g0 = plsc.load_gather(buf, [rowv, colv])
                g1 = plsc.load_gather(buf, [rowv, colv + 16])
                acc0 = g0 if acc0 is None else acc0 + g0
                acc1 = g1 if acc1 is None else acc1 + g1
            row = chunk * _CB + lb
            out_v[row, pl.ds(0, 16)] = acc0
            out_v[row, pl.ds(16, 16)] = acc1

    for k in range(_NBUF):
        start(k, k)

    @pl.loop(0, _NCHUNKS - _NBUF, step=_NBUF)
    def _(c):
        for k in range(_NBUF):
            chunk = c + k
            wait(chunk, k)
            accum(chunk, k)
            start(chunk + _NBUF, k)

    for k in range(_NBUF):
        chunk = _NCHUNKS - _NBUF + k
        wait(chunk, k)
        accum(chunk, k)

    pltpu.sync_copy(out_v, out_hbm.at[pl.ds(wid * _ROWS_PER_W, _ROWS_PER_W)])


@jax.jit
def _multi_embed(line_idx, colb, table_lines):
    mesh = plsc.VectorSubcoreMesh(
        core_axis_name="c", subcore_axis_name="s",
        num_cores=_NC, num_subcores=_NS)
    run = pl.kernel(
        _sc_body,
        out_type=jax.ShapeDtypeStruct((_B, _DIM), jnp.float32),
        mesh=mesh,
        scratch_types=[
            pltpu.VMEM((_NCHUNKS, _CHUNK_IDX), jnp.int32),
            [pltpu.VMEM((_CHUNK_IDX, 16), jnp.int32)
             for _ in range(_NBUF)],
            [pltpu.VMEM((_CHUNK_IDX, _LANE), jnp.float32)
             for _ in range(_NBUF)],
            pltpu.VMEM((_ROWS_PER_W, _DIM), jnp.float32),
            [pltpu.SemaphoreType.DMA for _ in range(_NBUF)],
        ],
        compiler_params=pltpu.CompilerParams(
            use_tc_tiling_on_sc=False, needs_layout_passes=False),
    )
    return run(line_idx, colb, table_lines)


def kernel(inputs, tables):
    table_lines = tables.reshape(_F * _VOCAB // _RPL, _LANE)
    offs = (jnp.arange(_F, dtype=jnp.int32) * _VOCAB)[None, :]
    idx = inputs.astype(jnp.int32) + offs
    line_idx = (idx // _RPL).reshape(_NW, _NCHUNKS, _CHUNK_IDX)
    colbase = ((idx % _RPL) * _DIM).reshape(_NW, _NCHUNKS, _CHUNK_IDX)
    colb = colbase[..., None] + jnp.arange(16, dtype=jnp.int32)
    return _multi_embed(line_idx, colb, table_lines)
